# Initial kernel scaffold; baseline (speedup 1.0000x reference)
import jax
import jax.numpy as jnp
from jax.experimental import pallas as pl


def kernel(Wqkv, Wo, x):
    raise NotImplementedError("write your pallas kernel here")


# speedup vs baseline: 3.7690x; 3.7690x over previous
import functools

import jax
import jax.numpy as jnp
from jax import lax
from jax.experimental import pallas as pl
from jax.experimental.pallas import tpu as pltpu


def _fused_kernel(x_ref, wqkv_ref, wo_ref, o_ref, *, num_heads, dk, dv, L,
                  eps, scale):
    nq = num_heads * dk
    x = x_ref[0]
    qkv = jnp.dot(x, wqkv_ref[...], preferred_element_type=jnp.float32)

    q = (qkv[:, :nq] * scale).astype(jnp.bfloat16)
    k = qkv[:, nq:2 * nq].astype(jnp.bfloat16)
    v = qkv[:, 2 * nq:].astype(jnp.bfloat16)

    row = lax.broadcasted_iota(jnp.int32, (L, L), 0)
    col = lax.broadcasted_iota(jnp.int32, (L, L), 1)
    causal = col <= row

    o_parts = []
    for h in range(num_heads):
        qh = q[:, h * dk:(h + 1) * dk]
        kh = k[:, h * dk:(h + 1) * dk]
        vh = v[:, h * dv:(h + 1) * dv]
        s = lax.dot_general(qh, kh, (((1,), (1,)), ((), ())),
                            preferred_element_type=jnp.float32)
        attn = 1.0 + s + 0.5 * (s * s)
        attn = jnp.where(causal, attn, 0.0)
        z = jnp.sum(attn, axis=-1, keepdims=True)
        oh = jnp.dot(attn.astype(jnp.bfloat16), vh,
                     preferred_element_type=jnp.float32)
        o_parts.append(oh * (1.0 / (z + eps)))
    o_norm = jnp.concatenate(o_parts, axis=-1).astype(jnp.bfloat16)

    o_ref[0] = jnp.dot(o_norm, wo_ref[...],
                       preferred_element_type=jnp.float32).astype(o_ref.dtype)


def kernel(Wqkv, Wo, x):
    B, L, D = x.shape
    num_heads = 8
    dk = 16
    nq = num_heads * dk
    nv = Wo.shape[0]
    dv = nv // num_heads
    eps = 1e-6
    scale = float(dk) ** -0.5

    xb = x.astype(jnp.bfloat16)
    wqkvb = Wqkv.astype(jnp.bfloat16)
    wob = Wo.astype(jnp.bfloat16)

    body = functools.partial(_fused_kernel, num_heads=num_heads, dk=dk, dv=dv,
                             L=L, eps=eps, scale=scale)
    return pl.pallas_call(
        body,
        out_shape=jax.ShapeDtypeStruct((B, L, D), x.dtype),
        grid_spec=pltpu.PrefetchScalarGridSpec(
            num_scalar_prefetch=0,
            grid=(B,),
            in_specs=[
                pl.BlockSpec((1, L, D), lambda b: (b, 0, 0)),
                pl.BlockSpec((D, 2 * nq + nv), lambda b: (0, 0)),
                pl.BlockSpec((nv, D), lambda b: (0, 0)),
            ],
            out_specs=pl.BlockSpec((1, L, D), lambda b: (b, 0, 0)),
        ),
        compiler_params=pltpu.CompilerParams(
            dimension_semantics=("parallel",)),
    )(xb, wqkvb, wob)


# speedup vs baseline: 4.4926x; 1.1920x over previous
import functools

import jax
import jax.numpy as jnp
from jax import lax
from jax.experimental import pallas as pl
from jax.experimental.pallas import tpu as pltpu


def _fused_kernel(x_ref, wqkv_ref, wo_ref, o_ref, *, num_heads, dk, dv, L,
                  eps, scale):
    nq = num_heads * dk
    x = x_ref[0].astype(jnp.bfloat16)
    qkv = jnp.dot(x, wqkv_ref[...], preferred_element_type=jnp.float32)

    q = (qkv[:, :nq] * scale).astype(jnp.bfloat16)
    k = qkv[:, nq:2 * nq].astype(jnp.bfloat16)
    v = qkv[:, 2 * nq:].astype(jnp.bfloat16)

    row = lax.broadcasted_iota(jnp.int32, (L, L), 0)
    col = lax.broadcasted_iota(jnp.int32, (L, L), 1)
    causal = col <= row

    o_parts = []
    for h in range(num_heads):
        qh = q[:, h * dk:(h + 1) * dk]
        kh = k[:, h * dk:(h + 1) * dk]
        vh = v[:, h * dv:(h + 1) * dv]
        s = lax.dot_general(qh, kh, (((1,), (1,)), ((), ())),
                            preferred_element_type=jnp.float32)
        attn = 1.0 + s + 0.5 * (s * s)
        attn = jnp.where(causal, attn, 0.0)
        z = jnp.sum(attn, axis=-1, keepdims=True)
        oh = jnp.dot(attn.astype(jnp.bfloat16), vh,
                     preferred_element_type=jnp.float32)
        o_parts.append(oh * (1.0 / (z + eps)))
    o_norm = jnp.concatenate(o_parts, axis=-1).astype(jnp.bfloat16)

    o_ref[0] = jnp.dot(o_norm, wo_ref[...],
                       preferred_element_type=jnp.float32).astype(o_ref.dtype)


def kernel(Wqkv, Wo, x):
    B, L, D = x.shape
    num_heads = 8
    dk = 16
    nq = num_heads * dk
    nv = Wo.shape[0]
    dv = nv // num_heads
    eps = 1e-6
    scale = float(dk) ** -0.5

    wqkvb = Wqkv.astype(jnp.bfloat16)
    wob = Wo.astype(jnp.bfloat16)

    body = functools.partial(_fused_kernel, num_heads=num_heads, dk=dk, dv=dv,
                             L=L, eps=eps, scale=scale)
    return pl.pallas_call(
        body,
        out_shape=jax.ShapeDtypeStruct((B, L, D), x.dtype),
        grid_spec=pltpu.PrefetchScalarGridSpec(
            num_scalar_prefetch=0,
            grid=(B,),
            in_specs=[
                pl.BlockSpec((1, L, D), lambda b: (b, 0, 0)),
                pl.BlockSpec((D, 2 * nq + nv), lambda b: (0, 0)),
                pl.BlockSpec((nv, D), lambda b: (0, 0)),
            ],
            out_specs=pl.BlockSpec((1, L, D), lambda b: (b, 0, 0)),
        ),
        compiler_params=pltpu.CompilerParams(
            dimension_semantics=("parallel",)),
    )(x, wqkvb, wob)


# speedup vs baseline: 4.6776x; 1.0412x over previous
import functools

import jax
import jax.numpy as jnp
from jax import lax
from jax.experimental import pallas as pl
from jax.experimental.pallas import tpu as pltpu


def _fused_kernel(x_ref, wqkv_ref, wo_ref, o_ref, wqkv_bf, wo_bf, *,
                  num_heads, dk, dv, L, eps, scale):
    nq = num_heads * dk

    @pl.when(pl.program_id(0) == 0)
    def _cast_weights():
        wqkv_bf[...] = wqkv_ref[...].astype(jnp.bfloat16)
        wo_bf[...] = wo_ref[...].astype(jnp.bfloat16)

    x = x_ref[0].astype(jnp.bfloat16)
    qkv = jnp.dot(x, wqkv_bf[...], preferred_element_type=jnp.float32)

    q = (qkv[:, :nq] * scale).astype(jnp.bfloat16)
    k = qkv[:, nq:2 * nq].astype(jnp.bfloat16)
    v = qkv[:, 2 * nq:].astype(jnp.bfloat16)

    row = lax.broadcasted_iota(jnp.int32, (L, L), 0)
    col = lax.broadcasted_iota(jnp.int32, (L, L), 1)
    causal = col <= row

    o_parts = []
    for h in range(num_heads):
        qh = q[:, h * dk:(h + 1) * dk]
        kh = k[:, h * dk:(h + 1) * dk]
        vh = v[:, h * dv:(h + 1) * dv]
        s = lax.dot_general(qh, kh, (((1,), (1,)), ((), ())),
                            preferred_element_type=jnp.float32)
        attn = 1.0 + s + 0.5 * (s * s)
        attn = jnp.where(causal, attn, 0.0)
        z = jnp.sum(attn, axis=-1, keepdims=True)
        oh = jnp.dot(attn.astype(jnp.bfloat16), vh,
                     preferred_element_type=jnp.float32)
        o_parts.append(oh * (1.0 / (z + eps)))
    o_norm = jnp.concatenate(o_parts, axis=-1).astype(jnp.bfloat16)

    o_ref[0] = jnp.dot(o_norm, wo_bf[...],
                       preferred_element_type=jnp.float32).astype(o_ref.dtype)


def kernel(Wqkv, Wo, x):
    B, L, D = x.shape
    num_heads = 8
    dk = 16
    nq = num_heads * dk
    nv = Wo.shape[0]
    dv = nv // num_heads
    eps = 1e-6
    scale = float(dk) ** -0.5

    body = functools.partial(_fused_kernel, num_heads=num_heads, dk=dk, dv=dv,
                             L=L, eps=eps, scale=scale)
    return pl.pallas_call(
        body,
        out_shape=jax.ShapeDtypeStruct((B, L, D), x.dtype),
        grid_spec=pltpu.PrefetchScalarGridSpec(
            num_scalar_prefetch=0,
            grid=(B,),
            in_specs=[
                pl.BlockSpec((1, L, D), lambda b: (b, 0, 0)),
                pl.BlockSpec((D, 2 * nq + nv), lambda b: (0, 0)),
                pl.BlockSpec((nv, D), lambda b: (0, 0)),
            ],
            out_specs=pl.BlockSpec((1, L, D), lambda b: (b, 0, 0)),
            scratch_shapes=[
                pltpu.VMEM((D, 2 * nq + nv), jnp.bfloat16),
                pltpu.VMEM((nv, D), jnp.bfloat16),
            ],
        ),
        compiler_params=pltpu.CompilerParams(
            dimension_semantics=("arbitrary",)),
    )(x, Wqkv, Wo)
